# baseline (speedup 1.0000x reference)
import functools
import math

import jax
import jax.numpy as jnp
from jax import lax
from jax.experimental import pallas as pl
from jax.experimental.pallas import tpu as pltpu
from jax.experimental.pallas import tpu_sc as plsc

N = 10000
E = 320000
HID = 128
G = 64
BN_EPS = 1e-5

NC = 2
NS = 16
NW = NC * NS
EPT = E // NW
K = 80
NCHUNK = EPT // K
NP = 10240
RPT = NP // NS
ZR = 128


def _sc_mesh():
    return plsc.VectorSubcoreMesh(core_axis_name="c", subcore_axis_name="s")


_SC_PARAMS = pltpu.CompilerParams(use_tc_tiling_on_sc=False)


def _zero_fill(buf, rows, w, dtype=jnp.float32):
    step = 16 if dtype == jnp.float32 else 32
    z = jnp.zeros((step,), dtype)

    def body(r, carry):
        for j in range(w // step):
            buf[r, pl.ds(j * step, step)] = z
        return carry

    lax.fori_loop(0, rows, body, 0, unroll=False)


def _make_agg_pass(w, nbuf, dtype=jnp.float32):
    ngrp = NCHUNK // nbuf
    ntail = NCHUNK - nbuf * ngrp

    @functools.partial(
        pl.kernel,
        mesh=_sc_mesh(),
        out_type=jax.ShapeDtypeStruct((NC * NP, w), dtype),
        compiler_params=_SC_PARAMS,
        scratch_types=[
            pltpu.VMEM((EPT,), jnp.int32),
            [pltpu.VMEM((K,), jnp.int32) for _ in range(nbuf)],
            [pltpu.VMEM((K, w), dtype) for _ in range(nbuf)],
            pltpu.VMEM_SHARED((NP, w), dtype),
            [pltpu.SemaphoreType.DMA for _ in range(nbuf)],
            [pltpu.SemaphoreType.DMA for _ in range(nbuf)],
            [pltpu.SemaphoreType.DMA for _ in range(nbuf)],
        ],
    )
    def agg_pass(y_hbm, src_hbm, dst_hbm, out_hbm,
                 sidx_all, didx, rows, acc, gsem, dsem, ssem):
        c = lax.axis_index("c")
        s = lax.axis_index("s")
        wid = c * NS + s
        ebase = wid * EPT
        pltpu.async_copy(src_hbm.at[pl.ds(ebase, EPT)], sidx_all, gsem[0])
        _zero_fill(rows[0], K, w, dtype)
        rbase = s * RPT
        for b in range(RPT // K):
            pltpu.sync_copy(rows[0], acc.at[pl.ds(rbase + b * K, K)])
        pltpu.make_async_copy(src_hbm.at[pl.ds(0, EPT)], sidx_all,
                              gsem[0]).wait()
        plsc.subcore_barrier()

        def gather(chunk, b):
            pltpu.async_copy(
                y_hbm.at[sidx_all.at[pl.ds(chunk * K, K)]], rows[b], gsem[b])

        def dload(chunk, b):
            pltpu.async_copy(
                dst_hbm.at[pl.ds(ebase + chunk * K, K)], didx[b], dsem[b])

        def wait_in(b):
            pltpu.make_async_copy(dst_hbm.at[pl.ds(0, K)], didx[b],
                                  dsem[b]).wait()
            pltpu.make_async_copy(y_hbm.at[sidx_all.at[pl.ds(0, K)]],
                                  rows[b], gsem[b]).wait()

        def scatter(b):
            pltpu.async_copy(rows[b], acc.at[didx[b]], ssem[b], add=True)

        def wait_scat(b):
            pltpu.make_async_copy(rows[b], acc.at[didx[b]], ssem[b]).wait()

        for b in range(nbuf):
            dload(b, b)
            gather(b, b)

        def group(j, carry):
            base = nbuf * j
            for b in range(nbuf):
                wait_in(b)
                scatter(b)
            for b in range(nbuf):
                wait_scat(b)
                dload(base + nbuf + b, b)
                gather(base + nbuf + b, b)
            return carry

        lax.fori_loop(0, ngrp - 1, group, 0, unroll=False)
        for b in range(nbuf):
            wait_in(b)
            scatter(b)
        for b in range(ntail):
            wait_scat(b)
            dload(nbuf * ngrp + b, b)
            gather(nbuf * ngrp + b, b)
        for b in range(ntail):
            wait_in(b)
            scatter(b)
        for b in range(nbuf):
            wait_scat(b)

        plsc.subcore_barrier()
        pltpu.sync_copy(acc.at[pl.ds(rbase, RPT)],
                        out_hbm.at[pl.ds(c * NP + rbase, RPT)])

    return agg_pass


def _make_p16_pass():
    nbuf = 8
    ngrp = NCHUNK // nbuf
    ntail = NCHUNK - nbuf * ngrp
    epta = E // NS
    nchunka = epta // K
    ngrp2 = nchunka // nbuf
    ntail2 = nchunka - nbuf * ngrp2

    @functools.partial(
        pl.kernel,
        mesh=_sc_mesh(),
        out_type=[jax.ShapeDtypeStruct((NC * NP, 16), jnp.float32),
                  jax.ShapeDtypeStruct((NP, 16), jnp.float32),
                  jax.ShapeDtypeStruct((NP, 16), jnp.float32)],
        compiler_params=_SC_PARAMS,
        scratch_types=[
            pltpu.VMEM((EPT,), jnp.int32),
            [pltpu.VMEM((K,), jnp.int32) for _ in range(nbuf)],
            [pltpu.VMEM((K, 16), jnp.float32) for _ in range(nbuf)],
            pltpu.VMEM((RPT, 16), jnp.float32),
            pltpu.VMEM((K, 16), jnp.float32),
            pltpu.VMEM((RPT, 16), jnp.float32),
            pltpu.VMEM((RPT, 16), jnp.float32),
            pltpu.VMEM((RPT, 16), jnp.float32),
            pltpu.VMEM_SHARED((NP, 16), jnp.float32),
            pltpu.VMEM_SHARED((NP, 16), jnp.float32),
            [pltpu.SemaphoreType.DMA for _ in range(nbuf)],
            [pltpu.SemaphoreType.DMA for _ in range(nbuf)],
            [pltpu.SemaphoreType.DMA for _ in range(nbuf)],
        ],
    )
    def p16_pass(xpad_hbm, src_hbm, dst_hbm,
                 s0_hbm, y0_hbm, dv_hbm,
                 sidx_all, didx, rows, pa, ones_v, px, py, pd,
                 ybuf, acc, gsem, dsem, ssem):
        c = lax.axis_index("c")
        s = lax.axis_index("s")
        wid = c * NS + s
        ebase = wid * EPT
        rbase = s * RPT
        pltpu.async_copy(src_hbm.at[pl.ds(ebase, EPT)], sidx_all, gsem[0])
        pltpu.async_copy(xpad_hbm.at[pl.ds(rbase, RPT)], px, gsem[3])
        _zero_fill(rows[0], K, 16)
        o16 = jnp.ones((16,), jnp.float32)

        def fill1(r, carry):
            ones_v[r, pl.ds(0, 16)] = o16
            return carry

        lax.fori_loop(0, K, fill1, 0, unroll=False)
        for b in range(RPT // K):
            pltpu.sync_copy(rows[0], acc.at[pl.ds(rbase + b * K, K)])
        plsc.subcore_barrier()

        dbase = s * epta

        def ddload(chunk, b):
            pltpu.async_copy(
                dst_hbm.at[pl.ds(dbase + chunk * K, K)], didx[b], dsem[b])

        def dwait_in(b):
            pltpu.make_async_copy(dst_hbm.at[pl.ds(0, K)], didx[b],
                                  dsem[b]).wait()

        def dscatter(b):
            pltpu.async_copy(ones_v, acc.at[didx[b]], ssem[b], add=True)

        def dwait_scat(b):
            pltpu.make_async_copy(ones_v, acc.at[didx[b]], ssem[b]).wait()

        for b in range(nbuf):
            ddload(b, b)

        def dgroup(j, carry):
            base = nbuf * j
            for b in range(nbuf):
                dwait_in(b)
                dscatter(b)
            for b in range(nbuf):
                dwait_scat(b)
                ddload(base + nbuf + b, b)
            return carry

        lax.fori_loop(0, ngrp2 - 1, dgroup, 0, unroll=False)
        for b in range(nbuf):
            dwait_in(b)
            dscatter(b)
        for b in range(ntail2):
            dwait_scat(b)
            ddload(nbuf * ngrp2 + b, b)
        for b in range(ntail2):
            dwait_in(b)
            dscatter(b)
        for b in range(nbuf):
            dwait_scat(b)
        plsc.subcore_barrier()

        pltpu.sync_copy(acc.at[pl.ds(rbase, RPT)], pa)
        pltpu.make_async_copy(xpad_hbm.at[pl.ds(0, RPT)], px, gsem[3]).wait()

        magic = jnp.full((16,), 0x5F3759DF, jnp.int32)
        one_i = jnp.full((16,), 1, jnp.int32)
        one_f = jnp.full((16,), 1.0, jnp.float32)
        half_f = jnp.full((16,), 0.5, jnp.float32)
        th_f = jnp.full((16,), 1.5, jnp.float32)

        def nrow(r, carry):
            d = pa[r, pl.ds(0, 16)] + one_f
            bits = lax.shift_right_arithmetic(
                lax.bitcast_convert_type(d, jnp.int32), one_i)
            y = lax.bitcast_convert_type(magic - bits, jnp.float32)
            hd = half_f * d
            for _ in range(3):
                y = y * (th_f - hd * y * y)
            pd[r, pl.ds(0, 16)] = y
            py[r, pl.ds(0, 16)] = y * px[r, pl.ds(0, 16)]
            return carry

        lax.fori_loop(0, RPT, nrow, 0, unroll=False)
        for b in range(RPT // K):
            pltpu.sync_copy(rows[0], acc.at[pl.ds(rbase + b * K, K)])
        pltpu.sync_copy(py, ybuf.at[pl.ds(rbase, RPT)])

        pltpu.sync_copy(py, y0_hbm.at[pl.ds(rbase, RPT)])
        pltpu.sync_copy(pd, dv_hbm.at[pl.ds(rbase, RPT)])

        pltpu.make_async_copy(src_hbm.at[pl.ds(0, EPT)], sidx_all,
                              gsem[0]).wait()
        plsc.subcore_barrier()

        def gather(chunk, b):
            pltpu.async_copy(
                ybuf.at[sidx_all.at[pl.ds(chunk * K, K)]], rows[b], gsem[b])

        def dload(chunk, b):
            pltpu.async_copy(
                dst_hbm.at[pl.ds(ebase + chunk * K, K)], didx[b], dsem[b])

        def wait_in(b):
            pltpu.make_async_copy(dst_hbm.at[pl.ds(0, K)], didx[b],
                                  dsem[b]).wait()
            pltpu.make_async_copy(ybuf.at[sidx_all.at[pl.ds(0, K)]],
                                  rows[b], gsem[b]).wait()

        def scatter(b):
            pltpu.async_copy(rows[b], acc.at[didx[b]], ssem[b], add=True)

        def wait_scat(b):
            pltpu.make_async_copy(rows[b], acc.at[didx[b]], ssem[b]).wait()

        for b in range(nbuf):
            dload(b, b)
            gather(b, b)

        def group(j, carry):
            base = nbuf * j
            for b in range(nbuf):
                wait_in(b)
                scatter(b)
            for b in range(nbuf):
                wait_scat(b)
                dload(base + nbuf + b, b)
                gather(base + nbuf + b, b)
            return carry

        lax.fori_loop(0, ngrp - 1, group, 0, unroll=False)
        for b in range(nbuf):
            wait_in(b)
            scatter(b)
        for b in range(ntail):
            wait_scat(b)
            dload(nbuf * ngrp + b, b)
            gather(nbuf * ngrp + b, b)
        for b in range(ntail):
            wait_in(b)
            scatter(b)
        for b in range(nbuf):
            wait_scat(b)

        plsc.subcore_barrier()
        pltpu.sync_copy(acc.at[pl.ds(rbase, RPT)],
                        s0_hbm.at[pl.ds(c * NP + rbase, RPT)])

    return p16_pass


_BLK = 2000
_GRID = N // _BLK
_BN_C = 1.0 / math.sqrt(1.0 + BN_EPS)


def _tc_layer(sa, sb, yprev, dinv16, wp, b, g, bt):
    w = yprev.shape[1]
    win = wp.shape[0]

    out_dtype = jnp.bfloat16

    def body(sa_ref, sb_ref, yp_ref, d_ref, w_ref, b_ref, g_ref, bt_ref,
             out_ref):
        d16 = d_ref[...]
        dcol = d16[:, 0:1]
        sab = (sa_ref[...].astype(jnp.float32) +
               sb_ref[...].astype(jnp.float32) +
               yp_ref[...].astype(jnp.float32))
        agg = sab * (d16 if w == 16 else jnp.broadcast_to(dcol, (_BLK, w)))
        h = jnp.dot(agg, w_ref[...], preferred_element_type=jnp.float32)
        h = (h + b_ref[...]) * (_BN_C * g_ref[...]) + bt_ref[...]
        h = jnp.maximum(h, 0.0)
        out_ref[...] = (h * jnp.broadcast_to(dcol, (_BLK, HID))).astype(
            out_dtype)

    specw = pl.BlockSpec((_BLK, w), lambda i: (i, 0))
    spec16 = pl.BlockSpec((_BLK, 16), lambda i: (i, 0))
    specW = pl.BlockSpec((win, HID), lambda i: (0, 0))
    spec1 = pl.BlockSpec((1, HID), lambda i: (0, 0))
    return pl.pallas_call(
        body,
        grid=(_GRID,),
        in_specs=[specw, specw, specw, spec16, specW, spec1, spec1, spec1],
        out_specs=pl.BlockSpec((_BLK, HID), lambda i: (i, 0)),
        out_shape=jax.ShapeDtypeStruct((N, HID), out_dtype),
    )(sa, sb, yprev, dinv16, wp, b, g, bt)


def _tc_final(sa, sb, yprev, dinv16, wp, b, g, bt, batch2d):
    def body(sa_ref, sb_ref, yp_ref, d_ref, w_ref, b_ref, g_ref, bt_ref,
             bat_ref, out_ref, sums, cnts):
        i = pl.program_id(0)

        @pl.when(i == 0)
        def _():
            sums[...] = jnp.zeros((G, HID), jnp.float32)
            cnts[...] = jnp.zeros((G, HID), jnp.float32)

        d16 = d_ref[...]
        dcol = d16[:, 0:1]
        agg = (sa_ref[...].astype(jnp.float32) +
               sb_ref[...].astype(jnp.float32) +
               yp_ref[...].astype(jnp.float32)) * jnp.broadcast_to(
            dcol, (_BLK, HID))
        h = jnp.dot(agg, w_ref[...], preferred_element_type=jnp.float32)
        h = (h + b_ref[...]) * (_BN_C * g_ref[...]) + bt_ref[...]
        h = jnp.maximum(h, 0.0)

        seg = lax.broadcasted_iota(jnp.int32, (_BLK, G), 1)
        mask = (jnp.broadcast_to(bat_ref[...], (_BLK, G)) == seg).astype(
            jnp.float32)
        dn = (((0,), (0,)), ((), ()))
        sums[...] += lax.dot_general(mask, h, dn,
                                     preferred_element_type=jnp.float32)
        cnts[...] += lax.dot_general(mask, jnp.ones((_BLK, HID), jnp.float32),
                                     dn, preferred_element_type=jnp.float32)

        @pl.when(i == _GRID - 1)
        def _():
            out_ref[...] = sums[...] / jnp.maximum(cnts[...], 1.0)

    specw = pl.BlockSpec((_BLK, HID), lambda i: (i, 0))
    spec16 = pl.BlockSpec((_BLK, 16), lambda i: (i, 0))
    specW = pl.BlockSpec((HID, HID), lambda i: (0, 0))
    spec1 = pl.BlockSpec((1, HID), lambda i: (0, 0))
    specb = pl.BlockSpec((_BLK, 1), lambda i: (i, 0))
    return pl.pallas_call(
        body,
        grid=(_GRID,),
        in_specs=[specw, specw, specw, spec16, specW, spec1, spec1, spec1,
                  specb],
        out_specs=pl.BlockSpec((G, HID), lambda i: (0, 0)),
        out_shape=jax.ShapeDtypeStruct((G, HID), jnp.float32),
        scratch_shapes=[pltpu.VMEM((G, HID), jnp.float32),
                        pltpu.VMEM((G, HID), jnp.float32)],
    )(sa, sb, yprev, dinv16, wp, b, g, bt, batch2d)


def kernel(x, edge_index, edge_attr, batch,
           W1, b1, g1, bt1, W2, b2, g2, bt2, W3, b3, g3, bt3):
    src = edge_index[0].astype(jnp.int32)
    dst = edge_index[1].astype(jnp.int32)
    xpad = jnp.concatenate(
        [jnp.concatenate([x, jnp.zeros((N, 16 - x.shape[1]), jnp.float32)],
                         axis=1),
         jnp.zeros((NP - N, 16), jnp.float32)], axis=0)
    W1p = jnp.concatenate(
        [W1, jnp.zeros((16 - W1.shape[0], HID), jnp.float32)], axis=0)
    batch2d = batch.astype(jnp.int32).reshape(N, 1)
    r1 = lambda v: v.reshape(1, HID)

    s0, y0pf, dinv16f = _make_p16_pass()(xpad, src, dst)
    y0p = y0pf[:N]
    dinv16 = dinv16f[:N]
    y1 = _tc_layer(s0[:N], s0[NP:NP + N], y0p, dinv16, W1p,
                   r1(b1), r1(g1), r1(bt1))

    s1 = _make_agg_pass(HID, 4, jnp.bfloat16)(y1, src, dst)
    y2 = _tc_layer(s1[:N], s1[NP:NP + N], y1, dinv16, W2,
                   r1(b2), r1(g2), r1(bt2))

    s2 = _make_agg_pass(HID, 4, jnp.bfloat16)(y2, src, dst)
    return _tc_final(s2[:N], s2[NP:NP + N], y2, dinv16, W3,
                     r1(b3), r1(g3), r1(bt3), batch2d)


# speedup vs baseline: 1.0356x; 1.0356x over previous
import functools
import math

import jax
import jax.numpy as jnp
from jax import lax
from jax.experimental import pallas as pl
from jax.experimental.pallas import tpu as pltpu
from jax.experimental.pallas import tpu_sc as plsc

N = 10000
E = 320000
HID = 128
G = 64
BN_EPS = 1e-5

NC = 2
NS = 16
NW = NC * NS
EPT = E // NW
K = 80
NCHUNK = EPT // K
NBUF = 8
_DEG_NGRP = NCHUNK // NBUF
_DEG_NTAIL = NCHUNK - NBUF * _DEG_NGRP
NP = 10240
RPT = NP // NS
ZR = 128


def _sc_mesh():
    return plsc.VectorSubcoreMesh(core_axis_name="c", subcore_axis_name="s")


_SC_PARAMS = pltpu.CompilerParams(use_tc_tiling_on_sc=False)


def _zero_fill(buf, rows, w, dtype=jnp.float32):
    step = 16 if dtype == jnp.float32 else 32
    z = jnp.zeros((step,), dtype)

    def body(r, carry):
        for j in range(w // step):
            buf[r, pl.ds(j * step, step)] = z
        return carry

    lax.fori_loop(0, rows, body, 0, unroll=False)


def _make_deg_pass():
    @functools.partial(
        pl.kernel,
        mesh=_sc_mesh(),
        out_type=jax.ShapeDtypeStruct((NC * NP, 16), jnp.float32),
        compiler_params=_SC_PARAMS,
        scratch_types=[
            [pltpu.VMEM((K,), jnp.int32) for _ in range(NBUF)],
            pltpu.VMEM((K, 16), jnp.float32),
            pltpu.VMEM((ZR, 16), jnp.float32),
            pltpu.VMEM_SHARED((NP, 16), jnp.float32),
            [pltpu.SemaphoreType.DMA for _ in range(NBUF)],
            [pltpu.SemaphoreType.DMA for _ in range(NBUF)],
        ],
    )
    def deg_pass(dst_hbm, out_hbm, didx, ones_v, zbuf, acc, dsem, ssem):
        c = lax.axis_index("c")
        s = lax.axis_index("s")
        wid = c * NS + s
        _zero_fill(zbuf, ZR, 16)
        o16 = jnp.ones((16,), jnp.float32)

        def fill1(r, carry):
            ones_v[r, pl.ds(0, 16)] = o16
            return carry

        lax.fori_loop(0, K, fill1, 0, unroll=False)
        rbase = s * RPT
        for b in range(RPT // ZR):
            pltpu.sync_copy(zbuf, acc.at[pl.ds(rbase + b * ZR, ZR)])
        plsc.subcore_barrier()

        ebase = wid * EPT

        def dload(chunk, b):
            pltpu.async_copy(
                dst_hbm.at[pl.ds(ebase + chunk * K, K)], didx[b], dsem[b])

        def wait_in(b):
            pltpu.make_async_copy(dst_hbm.at[pl.ds(0, K)], didx[b],
                                  dsem[b]).wait()

        def scatter(b):
            pltpu.async_copy(ones_v, acc.at[didx[b]], ssem[b], add=True)

        def wait_scat(b):
            pltpu.make_async_copy(ones_v, acc.at[didx[b]], ssem[b]).wait()

        for b in range(NBUF):
            dload(b, b)

        def group(j, carry):
            base = NBUF * j
            for b in range(NBUF):
                wait_in(b)
                scatter(b)
            for b in range(NBUF):
                wait_scat(b)
                dload(base + NBUF + b, b)
            return carry

        lax.fori_loop(0, _DEG_NGRP - 1, group, 0, unroll=False)
        for b in range(NBUF):
            wait_in(b)
            scatter(b)
        for b in range(_DEG_NTAIL):
            wait_scat(b)
            dload(NBUF * _DEG_NGRP + b, b)
        for b in range(_DEG_NTAIL):
            wait_in(b)
            scatter(b)
        for b in range(NBUF):
            wait_scat(b)

        plsc.subcore_barrier()
        pltpu.sync_copy(acc.at[pl.ds(rbase, RPT)],
                        out_hbm.at[pl.ds(c * NP + rbase, RPT)])

    return deg_pass


def _make_agg_pass(w, nbuf, dtype=jnp.float32):
    ngrp = NCHUNK // nbuf
    ntail = NCHUNK - nbuf * ngrp

    @functools.partial(
        pl.kernel,
        mesh=_sc_mesh(),
        out_type=jax.ShapeDtypeStruct((NC * NP, w), dtype),
        compiler_params=_SC_PARAMS,
        scratch_types=[
            pltpu.VMEM((EPT,), jnp.int32),
            [pltpu.VMEM((K,), jnp.int32) for _ in range(nbuf)],
            [pltpu.VMEM((K, w), dtype) for _ in range(nbuf)],
            pltpu.VMEM_SHARED((NP, w), dtype),
            [pltpu.SemaphoreType.DMA for _ in range(nbuf)],
            [pltpu.SemaphoreType.DMA for _ in range(nbuf)],
            [pltpu.SemaphoreType.DMA for _ in range(nbuf)],
        ],
    )
    def agg_pass(y_hbm, src_hbm, dst_hbm, out_hbm,
                 sidx_all, didx, rows, acc, gsem, dsem, ssem):
        c = lax.axis_index("c")
        s = lax.axis_index("s")
        wid = c * NS + s
        ebase = wid * EPT
        pltpu.async_copy(src_hbm.at[pl.ds(ebase, EPT)], sidx_all, gsem[0])
        _zero_fill(rows[0], K, w, dtype)
        rbase = s * RPT
        for b in range(RPT // K):
            pltpu.sync_copy(rows[0], acc.at[pl.ds(rbase + b * K, K)])
        pltpu.make_async_copy(src_hbm.at[pl.ds(0, EPT)], sidx_all,
                              gsem[0]).wait()
        plsc.subcore_barrier()

        def gather(chunk, b):
            pltpu.async_copy(
                y_hbm.at[sidx_all.at[pl.ds(chunk * K, K)]], rows[b], gsem[b])

        def dload(chunk, b):
            pltpu.async_copy(
                dst_hbm.at[pl.ds(ebase + chunk * K, K)], didx[b], dsem[b])

        def wait_in(b):
            pltpu.make_async_copy(dst_hbm.at[pl.ds(0, K)], didx[b],
                                  dsem[b]).wait()
            pltpu.make_async_copy(y_hbm.at[sidx_all.at[pl.ds(0, K)]],
                                  rows[b], gsem[b]).wait()

        def scatter(b):
            pltpu.async_copy(rows[b], acc.at[didx[b]], ssem[b], add=True)

        def wait_scat(b):
            pltpu.make_async_copy(rows[b], acc.at[didx[b]], ssem[b]).wait()

        for b in range(nbuf):
            dload(b, b)
            gather(b, b)

        def group(j, carry):
            base = nbuf * j
            for b in range(nbuf):
                wait_in(b)
                scatter(b)
            for b in range(nbuf):
                wait_scat(b)
                dload(base + nbuf + b, b)
                gather(base + nbuf + b, b)
            return carry

        lax.fori_loop(0, ngrp - 1, group, 0, unroll=False)
        for b in range(nbuf):
            wait_in(b)
            scatter(b)
        for b in range(ntail):
            wait_scat(b)
            dload(nbuf * ngrp + b, b)
            gather(nbuf * ngrp + b, b)
        for b in range(ntail):
            wait_in(b)
            scatter(b)
        for b in range(nbuf):
            wait_scat(b)

        plsc.subcore_barrier()
        pltpu.sync_copy(acc.at[pl.ds(rbase, RPT)],
                        out_hbm.at[pl.ds(c * NP + rbase, RPT)])

    return agg_pass


def _make_p16_pass():
    nbuf = 8
    ngrp = NCHUNK // nbuf
    ntail = NCHUNK - nbuf * ngrp

    @functools.partial(
        pl.kernel,
        mesh=_sc_mesh(),
        out_type=[jax.ShapeDtypeStruct((NC * NP, 16), jnp.float32),
                  jax.ShapeDtypeStruct((NP, 16), jnp.float32),
                  jax.ShapeDtypeStruct((NP, 16), jnp.float32)],
        compiler_params=_SC_PARAMS,
        scratch_types=[
            pltpu.VMEM((EPT,), jnp.int32),
            [pltpu.VMEM((K,), jnp.int32) for _ in range(nbuf)],
            [pltpu.VMEM((K, 16), jnp.float32) for _ in range(nbuf)],
            pltpu.VMEM((RPT, 16), jnp.float32),
            pltpu.VMEM((RPT, 16), jnp.float32),
            pltpu.VMEM((RPT, 16), jnp.float32),
            pltpu.VMEM((RPT, 16), jnp.float32),
            pltpu.VMEM((RPT, 16), jnp.float32),
            pltpu.VMEM_SHARED((NP, 16), jnp.float32),
            pltpu.VMEM_SHARED((NP, 16), jnp.float32),
            [pltpu.SemaphoreType.DMA for _ in range(nbuf)],
            [pltpu.SemaphoreType.DMA for _ in range(nbuf)],
            [pltpu.SemaphoreType.DMA for _ in range(nbuf)],
        ],
    )
    def p16_pass(degp_hbm, xpad_hbm, src_hbm, dst_hbm,
                 s0_hbm, y0_hbm, dv_hbm,
                 sidx_all, didx, rows, pa, pb, px, py, pd,
                 ybuf, acc, gsem, dsem, ssem):
        c = lax.axis_index("c")
        s = lax.axis_index("s")
        wid = c * NS + s
        ebase = wid * EPT
        rbase = s * RPT
        pltpu.async_copy(src_hbm.at[pl.ds(ebase, EPT)], sidx_all, gsem[0])
        pltpu.async_copy(degp_hbm.at[pl.ds(rbase, RPT)], pa, gsem[1])
        pltpu.async_copy(degp_hbm.at[pl.ds(NP + rbase, RPT)], pb, gsem[2])
        pltpu.async_copy(xpad_hbm.at[pl.ds(rbase, RPT)], px, gsem[3])
        _zero_fill(rows[0], K, 16)
        for b in range(RPT // K):
            pltpu.sync_copy(rows[0], acc.at[pl.ds(rbase + b * K, K)])
        pltpu.make_async_copy(degp_hbm.at[pl.ds(0, RPT)], pa, gsem[1]).wait()
        pltpu.make_async_copy(degp_hbm.at[pl.ds(0, RPT)], pb, gsem[2]).wait()
        pltpu.make_async_copy(xpad_hbm.at[pl.ds(0, RPT)], px, gsem[3]).wait()

        magic = jnp.full((16,), 0x5F3759DF, jnp.int32)
        one_i = jnp.full((16,), 1, jnp.int32)
        one_f = jnp.full((16,), 1.0, jnp.float32)
        half_f = jnp.full((16,), 0.5, jnp.float32)
        th_f = jnp.full((16,), 1.5, jnp.float32)

        def nrow(r, carry):
            d = pa[r, pl.ds(0, 16)] + pb[r, pl.ds(0, 16)] + one_f
            bits = lax.shift_right_arithmetic(
                lax.bitcast_convert_type(d, jnp.int32), one_i)
            y = lax.bitcast_convert_type(magic - bits, jnp.float32)
            hd = half_f * d
            for _ in range(3):
                y = y * (th_f - hd * y * y)
            pd[r, pl.ds(0, 16)] = y
            py[r, pl.ds(0, 16)] = y * px[r, pl.ds(0, 16)]
            return carry

        lax.fori_loop(0, RPT, nrow, 0, unroll=False)
        pltpu.sync_copy(py, ybuf.at[pl.ds(rbase, RPT)])

        pltpu.sync_copy(py, y0_hbm.at[pl.ds(rbase, RPT)])
        pltpu.sync_copy(pd, dv_hbm.at[pl.ds(rbase, RPT)])

        pltpu.make_async_copy(src_hbm.at[pl.ds(0, EPT)], sidx_all,
                              gsem[0]).wait()
        plsc.subcore_barrier()

        def gather(chunk, b):
            pltpu.async_copy(
                ybuf.at[sidx_all.at[pl.ds(chunk * K, K)]], rows[b], gsem[b])

        def dload(chunk, b):
            pltpu.async_copy(
                dst_hbm.at[pl.ds(ebase + chunk * K, K)], didx[b], dsem[b])

        def wait_in(b):
            pltpu.make_async_copy(dst_hbm.at[pl.ds(0, K)], didx[b],
                                  dsem[b]).wait()
            pltpu.make_async_copy(ybuf.at[sidx_all.at[pl.ds(0, K)]],
                                  rows[b], gsem[b]).wait()

        def scatter(b):
            pltpu.async_copy(rows[b], acc.at[didx[b]], ssem[b], add=True)

        def wait_scat(b):
            pltpu.make_async_copy(rows[b], acc.at[didx[b]], ssem[b]).wait()

        for b in range(nbuf):
            dload(b, b)
            gather(b, b)

        def group(j, carry):
            base = nbuf * j
            for b in range(nbuf):
                wait_in(b)
                scatter(b)
            for b in range(nbuf):
                wait_scat(b)
                dload(base + nbuf + b, b)
                gather(base + nbuf + b, b)
            return carry

        lax.fori_loop(0, ngrp - 1, group, 0, unroll=False)
        for b in range(nbuf):
            wait_in(b)
            scatter(b)
        for b in range(ntail):
            wait_scat(b)
            dload(nbuf * ngrp + b, b)
            gather(nbuf * ngrp + b, b)
        for b in range(ntail):
            wait_in(b)
            scatter(b)
        for b in range(nbuf):
            wait_scat(b)

        plsc.subcore_barrier()
        pltpu.sync_copy(acc.at[pl.ds(rbase, RPT)],
                        s0_hbm.at[pl.ds(c * NP + rbase, RPT)])

    return p16_pass


_BLK = 2000
_GRID = N // _BLK
_BN_C = 1.0 / math.sqrt(1.0 + BN_EPS)


def _tc_layer(sa, sb, yprev, dinv16, wp, b, g, bt):
    w = yprev.shape[1]
    win = wp.shape[0]

    out_dtype = jnp.bfloat16

    def body(sa_ref, sb_ref, yp_ref, d_ref, w_ref, b_ref, g_ref, bt_ref,
             out_ref):
        d16 = d_ref[...]
        dcol = d16[:, 0:1]
        sab = (sa_ref[...].astype(jnp.float32) +
               sb_ref[...].astype(jnp.float32) +
               yp_ref[...].astype(jnp.float32))
        agg = sab * (d16 if w == 16 else jnp.broadcast_to(dcol, (_BLK, w)))
        h = jnp.dot(agg, w_ref[...], preferred_element_type=jnp.float32)
        h = (h + b_ref[...]) * (_BN_C * g_ref[...]) + bt_ref[...]
        h = jnp.maximum(h, 0.0)
        out_ref[...] = (h * jnp.broadcast_to(dcol, (_BLK, HID))).astype(
            out_dtype)

    specw = pl.BlockSpec((_BLK, w), lambda i: (i, 0))
    spec16 = pl.BlockSpec((_BLK, 16), lambda i: (i, 0))
    specW = pl.BlockSpec((win, HID), lambda i: (0, 0))
    spec1 = pl.BlockSpec((1, HID), lambda i: (0, 0))
    return pl.pallas_call(
        body,
        grid=(_GRID,),
        in_specs=[specw, specw, specw, spec16, specW, spec1, spec1, spec1],
        out_specs=pl.BlockSpec((_BLK, HID), lambda i: (i, 0)),
        out_shape=jax.ShapeDtypeStruct((N, HID), out_dtype),
    )(sa, sb, yprev, dinv16, wp, b, g, bt)


def _tc_final(sa, sb, yprev, dinv16, wp, b, g, bt, batch2d):
    def body(sa_ref, sb_ref, yp_ref, d_ref, w_ref, b_ref, g_ref, bt_ref,
             bat_ref, out_ref, sums, cnts):
        i = pl.program_id(0)

        @pl.when(i == 0)
        def _():
            sums[...] = jnp.zeros((G, HID), jnp.float32)
            cnts[...] = jnp.zeros((G, HID), jnp.float32)

        d16 = d_ref[...]
        dcol = d16[:, 0:1]
        agg = (sa_ref[...].astype(jnp.float32) +
               sb_ref[...].astype(jnp.float32) +
               yp_ref[...].astype(jnp.float32)) * jnp.broadcast_to(
            dcol, (_BLK, HID))
        h = jnp.dot(agg, w_ref[...], preferred_element_type=jnp.float32)
        h = (h + b_ref[...]) * (_BN_C * g_ref[...]) + bt_ref[...]
        h = jnp.maximum(h, 0.0)

        seg = lax.broadcasted_iota(jnp.int32, (_BLK, G), 1)
        mask = (jnp.broadcast_to(bat_ref[...], (_BLK, G)) == seg).astype(
            jnp.float32)
        dn = (((0,), (0,)), ((), ()))
        sums[...] += lax.dot_general(mask, h, dn,
                                     preferred_element_type=jnp.float32)
        cnts[...] += lax.dot_general(mask, jnp.ones((_BLK, HID), jnp.float32),
                                     dn, preferred_element_type=jnp.float32)

        @pl.when(i == _GRID - 1)
        def _():
            out_ref[...] = sums[...] / jnp.maximum(cnts[...], 1.0)

    specw = pl.BlockSpec((_BLK, HID), lambda i: (i, 0))
    spec16 = pl.BlockSpec((_BLK, 16), lambda i: (i, 0))
    specW = pl.BlockSpec((HID, HID), lambda i: (0, 0))
    spec1 = pl.BlockSpec((1, HID), lambda i: (0, 0))
    specb = pl.BlockSpec((_BLK, 1), lambda i: (i, 0))
    return pl.pallas_call(
        body,
        grid=(_GRID,),
        in_specs=[specw, specw, specw, spec16, specW, spec1, spec1, spec1,
                  specb],
        out_specs=pl.BlockSpec((G, HID), lambda i: (0, 0)),
        out_shape=jax.ShapeDtypeStruct((G, HID), jnp.float32),
        scratch_shapes=[pltpu.VMEM((G, HID), jnp.float32),
                        pltpu.VMEM((G, HID), jnp.float32)],
    )(sa, sb, yprev, dinv16, wp, b, g, bt, batch2d)


def kernel(x, edge_index, edge_attr, batch,
           W1, b1, g1, bt1, W2, b2, g2, bt2, W3, b3, g3, bt3):
    src = edge_index[0].astype(jnp.int32)
    dst = edge_index[1].astype(jnp.int32)
    xpad = jnp.concatenate(
        [jnp.concatenate([x, jnp.zeros((N, 16 - x.shape[1]), jnp.float32)],
                         axis=1),
         jnp.zeros((NP - N, 16), jnp.float32)], axis=0)
    W1p = jnp.concatenate(
        [W1, jnp.zeros((16 - W1.shape[0], HID), jnp.float32)], axis=0)
    batch2d = batch.astype(jnp.int32).reshape(N, 1)
    r1 = lambda v: v.reshape(1, HID)

    degp = _make_deg_pass()(dst)
    s0, y0pf, dinv16f = _make_p16_pass()(degp, xpad, src, dst)
    y0p = y0pf[:N]
    dinv16 = dinv16f[:N]
    y1 = _tc_layer(s0[:N], s0[NP:NP + N], y0p, dinv16, W1p,
                   r1(b1), r1(g1), r1(bt1))

    s1 = _make_agg_pass(HID, 4, jnp.bfloat16)(y1, src, dst)
    y2 = _tc_layer(s1[:N], s1[NP:NP + N], y1, dinv16, W2,
                   r1(b2), r1(g2), r1(bt2))

    s2 = _make_agg_pass(HID, 4, jnp.bfloat16)(y2, src, dst)
    return _tc_final(s2[:N], s2[NP:NP + N], y2, dinv16, W3,
                     r1(b3), r1(g3), r1(bt3), batch2d)


# speedup vs baseline: 1.0578x; 1.0214x over previous
import functools
import math

import jax
import jax.numpy as jnp
from jax import lax
from jax.experimental import pallas as pl
from jax.experimental.pallas import tpu as pltpu
from jax.experimental.pallas import tpu_sc as plsc

N = 10000
E = 320000
HID = 128
G = 64
BN_EPS = 1e-5

NC = 2
NS = 16
NW = NC * NS
EPT = E // NW
K = 80
NCHUNK = EPT // K
NBUF = 8
_DEG_NGRP = NCHUNK // NBUF
_DEG_NTAIL = NCHUNK - NBUF * _DEG_NGRP
NP = 10240
RPT = NP // NS
ZR = 128


def _sc_mesh():
    return plsc.VectorSubcoreMesh(core_axis_name="c", subcore_axis_name="s")


_SC_PARAMS = pltpu.CompilerParams(use_tc_tiling_on_sc=False)


def _zero_fill(buf, rows, w, dtype=jnp.float32):
    step = 16 if dtype == jnp.float32 else 32
    z = jnp.zeros((step,), dtype)

    def body(r, carry):
        for j in range(w // step):
            buf[r, pl.ds(j * step, step)] = z
        return carry

    lax.fori_loop(0, rows, body, 0, unroll=False)


def _make_deg_pass():
    @functools.partial(
        pl.kernel,
        mesh=_sc_mesh(),
        out_type=jax.ShapeDtypeStruct((NC * NP, 16), jnp.float32),
        compiler_params=_SC_PARAMS,
        scratch_types=[
            [pltpu.VMEM((K,), jnp.int32) for _ in range(NBUF)],
            pltpu.VMEM((K, 16), jnp.float32),
            pltpu.VMEM((ZR, 16), jnp.float32),
            pltpu.VMEM_SHARED((NP, 16), jnp.float32),
            [pltpu.SemaphoreType.DMA for _ in range(NBUF)],
            [pltpu.SemaphoreType.DMA for _ in range(NBUF)],
        ],
    )
    def deg_pass(dst_hbm, out_hbm, didx, ones_v, zbuf, acc, dsem, ssem):
        c = lax.axis_index("c")
        s = lax.axis_index("s")
        wid = c * NS + s
        _zero_fill(zbuf, ZR, 16)
        o16 = jnp.ones((16,), jnp.float32)

        def fill1(r, carry):
            ones_v[r, pl.ds(0, 16)] = o16
            return carry

        lax.fori_loop(0, K, fill1, 0, unroll=False)
        rbase = s * RPT
        for b in range(RPT // ZR):
            pltpu.sync_copy(zbuf, acc.at[pl.ds(rbase + b * ZR, ZR)])
        plsc.subcore_barrier()

        ebase = wid * EPT

        def dload(chunk, b):
            pltpu.async_copy(
                dst_hbm.at[pl.ds(ebase + chunk * K, K)], didx[b], dsem[b])

        def wait_in(b):
            pltpu.make_async_copy(dst_hbm.at[pl.ds(0, K)], didx[b],
                                  dsem[b]).wait()

        def scatter(b):
            pltpu.async_copy(ones_v, acc.at[didx[b]], ssem[b], add=True)

        def wait_scat(b):
            pltpu.make_async_copy(ones_v, acc.at[didx[b]], ssem[b]).wait()

        for b in range(NBUF):
            dload(b, b)

        def group(j, carry):
            base = NBUF * j
            for b in range(NBUF):
                wait_in(b)
                scatter(b)
            for b in range(NBUF):
                wait_scat(b)
                dload(base + NBUF + b, b)
            return carry

        lax.fori_loop(0, _DEG_NGRP - 1, group, 0, unroll=False)
        for b in range(NBUF):
            wait_in(b)
            scatter(b)
        for b in range(_DEG_NTAIL):
            wait_scat(b)
            dload(NBUF * _DEG_NGRP + b, b)
        for b in range(_DEG_NTAIL):
            wait_in(b)
            scatter(b)
        for b in range(NBUF):
            wait_scat(b)

        plsc.subcore_barrier()
        pltpu.sync_copy(acc.at[pl.ds(rbase, RPT)],
                        out_hbm.at[pl.ds(c * NP + rbase, RPT)])

    return deg_pass


def _make_agg_pass(w, nbuf, dtype=jnp.float32):
    ngrp = NCHUNK // nbuf
    ntail = NCHUNK - nbuf * ngrp

    @functools.partial(
        pl.kernel,
        mesh=_sc_mesh(),
        out_type=jax.ShapeDtypeStruct((NC * NP, w), dtype),
        compiler_params=_SC_PARAMS,
        scratch_types=[
            pltpu.VMEM((EPT,), jnp.int32),
            [pltpu.VMEM((K,), jnp.int32) for _ in range(nbuf)],
            [pltpu.VMEM((K, w), dtype) for _ in range(nbuf)],
            pltpu.VMEM_SHARED((NP, w), dtype),
            [pltpu.SemaphoreType.DMA for _ in range(nbuf)],
            [pltpu.SemaphoreType.DMA for _ in range(nbuf)],
            [pltpu.SemaphoreType.DMA for _ in range(nbuf)],
        ],
    )
    def agg_pass(y_hbm, src_hbm, dst_hbm, out_hbm,
                 sidx_all, didx, rows, acc, gsem, dsem, ssem):
        c = lax.axis_index("c")
        s = lax.axis_index("s")
        wid = c * NS + s
        ebase = wid * EPT
        pltpu.async_copy(src_hbm.at[pl.ds(ebase, EPT)], sidx_all, gsem[0])
        _zero_fill(rows[0], K, w, dtype)
        rbase = s * RPT
        for b in range(RPT // K):
            pltpu.sync_copy(rows[0], acc.at[pl.ds(rbase + b * K, K)])
        pltpu.make_async_copy(src_hbm.at[pl.ds(0, EPT)], sidx_all,
                              gsem[0]).wait()
        plsc.subcore_barrier()

        def gather(chunk, b):
            pltpu.async_copy(
                y_hbm.at[sidx_all.at[pl.ds(chunk * K, K)]], rows[b], gsem[b])

        def dload(chunk, b):
            pltpu.async_copy(
                dst_hbm.at[pl.ds(ebase + chunk * K, K)], didx[b], dsem[b])

        def wait_in(b):
            pltpu.make_async_copy(dst_hbm.at[pl.ds(0, K)], didx[b],
                                  dsem[b]).wait()
            pltpu.make_async_copy(y_hbm.at[sidx_all.at[pl.ds(0, K)]],
                                  rows[b], gsem[b]).wait()

        def scatter(b):
            pltpu.async_copy(rows[b], acc.at[didx[b]], ssem[b], add=True)

        def wait_scat(b):
            pltpu.make_async_copy(rows[b], acc.at[didx[b]], ssem[b]).wait()

        for b in range(nbuf):
            dload(b, b)
            gather(b, b)

        def group(j, carry):
            base = nbuf * j
            for b in range(nbuf):
                wait_in(b)
                scatter(b)
            for b in range(nbuf):
                wait_scat(b)
                dload(base + nbuf + b, b)
                gather(base + nbuf + b, b)
            return carry

        lax.fori_loop(0, ngrp - 1, group, 0, unroll=False)
        for b in range(nbuf):
            wait_in(b)
            scatter(b)
        for b in range(ntail):
            wait_scat(b)
            dload(nbuf * ngrp + b, b)
            gather(nbuf * ngrp + b, b)
        for b in range(ntail):
            wait_in(b)
            scatter(b)
        for b in range(nbuf):
            wait_scat(b)

        plsc.subcore_barrier()
        pltpu.sync_copy(acc.at[pl.ds(rbase, RPT)],
                        out_hbm.at[pl.ds(c * NP + rbase, RPT)])

    return agg_pass


def _make_p16_pass():
    nbuf = 8
    ngrp = NCHUNK // nbuf
    ntail = NCHUNK - nbuf * ngrp

    @functools.partial(
        pl.kernel,
        mesh=_sc_mesh(),
        out_type=[jax.ShapeDtypeStruct((NC * NP, 16), jnp.float32),
                  jax.ShapeDtypeStruct((NP, 16), jnp.float32),
                  jax.ShapeDtypeStruct((NP, 16), jnp.float32)],
        compiler_params=_SC_PARAMS,
        scratch_types=[
            pltpu.VMEM((EPT,), jnp.int32),
            [pltpu.VMEM((K,), jnp.int32) for _ in range(nbuf)],
            [pltpu.VMEM((K, 16), jnp.float32) for _ in range(nbuf)],
            pltpu.VMEM((RPT, 16), jnp.float32),
            pltpu.VMEM((RPT, 16), jnp.float32),
            pltpu.VMEM((RPT, 16), jnp.float32),
            pltpu.VMEM((RPT, 16), jnp.float32),
            pltpu.VMEM((RPT, 16), jnp.float32),
            pltpu.VMEM_SHARED((NP, 16), jnp.float32),
            pltpu.VMEM_SHARED((NP, 16), jnp.float32),
            [pltpu.SemaphoreType.DMA for _ in range(nbuf)],
            [pltpu.SemaphoreType.DMA for _ in range(nbuf)],
            [pltpu.SemaphoreType.DMA for _ in range(nbuf)],
        ],
    )
    def p16_pass(degp_hbm, xpad_hbm, src_hbm, dst_hbm,
                 s0_hbm, y0_hbm, dv_hbm,
                 sidx_all, didx, rows, pa, pb, px, py, pd,
                 ybuf, acc, gsem, dsem, ssem):
        c = lax.axis_index("c")
        s = lax.axis_index("s")
        wid = c * NS + s
        ebase = wid * EPT
        rbase = s * RPT
        pltpu.async_copy(src_hbm.at[pl.ds(ebase, EPT)], sidx_all, gsem[0])
        pltpu.async_copy(degp_hbm.at[pl.ds(rbase, RPT)], pa, gsem[1])
        pltpu.async_copy(degp_hbm.at[pl.ds(NP + rbase, RPT)], pb, gsem[2])
        pltpu.async_copy(xpad_hbm.at[pl.ds(rbase, RPT)], px, gsem[3])
        _zero_fill(rows[0], K, 16)
        for b in range(RPT // K):
            pltpu.sync_copy(rows[0], acc.at[pl.ds(rbase + b * K, K)])
        pltpu.make_async_copy(degp_hbm.at[pl.ds(0, RPT)], pa, gsem[1]).wait()
        pltpu.make_async_copy(degp_hbm.at[pl.ds(0, RPT)], pb, gsem[2]).wait()
        pltpu.make_async_copy(xpad_hbm.at[pl.ds(0, RPT)], px, gsem[3]).wait()

        magic = jnp.full((16,), 0x5F3759DF, jnp.int32)
        one_i = jnp.full((16,), 1, jnp.int32)
        one_f = jnp.full((16,), 1.0, jnp.float32)
        half_f = jnp.full((16,), 0.5, jnp.float32)
        th_f = jnp.full((16,), 1.5, jnp.float32)

        def nrow(r, carry):
            d = pa[r, pl.ds(0, 16)] + pb[r, pl.ds(0, 16)] + one_f
            bits = lax.shift_right_arithmetic(
                lax.bitcast_convert_type(d, jnp.int32), one_i)
            y = lax.bitcast_convert_type(magic - bits, jnp.float32)
            hd = half_f * d
            for _ in range(3):
                y = y * (th_f - hd * y * y)
            pd[r, pl.ds(0, 16)] = y
            py[r, pl.ds(0, 16)] = y * px[r, pl.ds(0, 16)]
            return carry

        lax.fori_loop(0, RPT, nrow, 0, unroll=False)
        pltpu.sync_copy(py, ybuf.at[pl.ds(rbase, RPT)])

        pltpu.sync_copy(py, y0_hbm.at[pl.ds(rbase, RPT)])
        pltpu.sync_copy(pd, dv_hbm.at[pl.ds(rbase, RPT)])

        pltpu.make_async_copy(src_hbm.at[pl.ds(0, EPT)], sidx_all,
                              gsem[0]).wait()
        plsc.subcore_barrier()

        def gather(chunk, b):
            pltpu.async_copy(
                ybuf.at[sidx_all.at[pl.ds(chunk * K, K)]], rows[b], gsem[b])

        def dload(chunk, b):
            pltpu.async_copy(
                dst_hbm.at[pl.ds(ebase + chunk * K, K)], didx[b], dsem[b])

        def wait_in(b):
            pltpu.make_async_copy(dst_hbm.at[pl.ds(0, K)], didx[b],
                                  dsem[b]).wait()
            pltpu.make_async_copy(ybuf.at[sidx_all.at[pl.ds(0, K)]],
                                  rows[b], gsem[b]).wait()

        def scatter(b):
            pltpu.async_copy(rows[b], acc.at[didx[b]], ssem[b], add=True)

        def wait_scat(b):
            pltpu.make_async_copy(rows[b], acc.at[didx[b]], ssem[b]).wait()

        for b in range(nbuf):
            dload(b, b)
            gather(b, b)

        def group(j, carry):
            base = nbuf * j
            for b in range(nbuf):
                wait_in(b)
                scatter(b)
            for b in range(nbuf):
                wait_scat(b)
                dload(base + nbuf + b, b)
                gather(base + nbuf + b, b)
            return carry

        lax.fori_loop(0, ngrp - 1, group, 0, unroll=False)
        for b in range(nbuf):
            wait_in(b)
            scatter(b)
        for b in range(ntail):
            wait_scat(b)
            dload(nbuf * ngrp + b, b)
            gather(nbuf * ngrp + b, b)
        for b in range(ntail):
            wait_in(b)
            scatter(b)
        for b in range(nbuf):
            wait_scat(b)

        plsc.subcore_barrier()
        pltpu.sync_copy(acc.at[pl.ds(rbase, RPT)],
                        s0_hbm.at[pl.ds(c * NP + rbase, RPT)])

    return p16_pass


_BLK = 2000
_GRID = N // _BLK
_BN_C = 1.0 / math.sqrt(1.0 + BN_EPS)


def _tc_layer(sa, sb, yprev, dinv16, wp, b, g, bt):
    w = yprev.shape[1]
    win = wp.shape[0]

    out_dtype = jnp.bfloat16

    def body(sa_ref, sb_ref, yp_ref, d_ref, w_ref, b_ref, g_ref, bt_ref,
             out_ref):
        d16 = d_ref[...]
        dcol = d16[:, 0:1]
        sab = (sa_ref[...].astype(jnp.float32) +
               sb_ref[...].astype(jnp.float32) +
               yp_ref[...].astype(jnp.float32))
        agg = sab * (d16 if w == 16 else jnp.broadcast_to(dcol, (_BLK, w)))
        h = jnp.dot(agg, w_ref[...], preferred_element_type=jnp.float32)
        h = (h + b_ref[...]) * (_BN_C * g_ref[...]) + bt_ref[...]
        h = jnp.maximum(h, 0.0)
        out_ref[...] = (h * jnp.broadcast_to(dcol, (_BLK, HID))).astype(
            out_dtype)

    specw = pl.BlockSpec((_BLK, w), lambda i: (i, 0))
    spec16 = pl.BlockSpec((_BLK, 16), lambda i: (i, 0))
    specW = pl.BlockSpec((win, HID), lambda i: (0, 0))
    spec1 = pl.BlockSpec((1, HID), lambda i: (0, 0))
    return pl.pallas_call(
        body,
        grid=(_GRID,),
        in_specs=[specw, specw, specw, spec16, specW, spec1, spec1, spec1],
        out_specs=pl.BlockSpec((_BLK, HID), lambda i: (i, 0)),
        out_shape=jax.ShapeDtypeStruct((N, HID), out_dtype),
    )(sa, sb, yprev, dinv16, wp, b, g, bt)


def _tc_final(sa, sb, yprev, dinv16, wp, b, g, bt, batch2d):
    def body(sa_ref, sb_ref, yp_ref, d_ref, w_ref, b_ref, g_ref, bt_ref,
             bat_ref, out_ref, sums, cnts):
        i = pl.program_id(0)

        @pl.when(i == 0)
        def _():
            sums[...] = jnp.zeros((G, HID), jnp.float32)
            cnts[...] = jnp.zeros((G, HID), jnp.float32)

        d16 = d_ref[...]
        dcol = d16[:, 0:1]
        agg = (sa_ref[...].astype(jnp.float32) +
               sb_ref[...].astype(jnp.float32) +
               yp_ref[...].astype(jnp.float32)) * jnp.broadcast_to(
            dcol, (_BLK, HID))
        h = jnp.dot(agg, w_ref[...], preferred_element_type=jnp.float32)
        h = (h + b_ref[...]) * (_BN_C * g_ref[...]) + bt_ref[...]
        h = jnp.maximum(h, 0.0)

        seg = lax.broadcasted_iota(jnp.int32, (_BLK, G), 1)
        mask = (jnp.broadcast_to(bat_ref[...], (_BLK, G)) == seg).astype(
            jnp.float32)
        dn = (((0,), (0,)), ((), ()))
        sums[...] += lax.dot_general(mask, h, dn,
                                     preferred_element_type=jnp.float32)
        cnts[...] += lax.dot_general(mask, jnp.ones((_BLK, HID), jnp.float32),
                                     dn, preferred_element_type=jnp.float32)

        @pl.when(i == _GRID - 1)
        def _():
            out_ref[...] = sums[...] / jnp.maximum(cnts[...], 1.0)

    specw = pl.BlockSpec((_BLK, HID), lambda i: (i, 0))
    spec16 = pl.BlockSpec((_BLK, 16), lambda i: (i, 0))
    specW = pl.BlockSpec((HID, HID), lambda i: (0, 0))
    spec1 = pl.BlockSpec((1, HID), lambda i: (0, 0))
    specb = pl.BlockSpec((_BLK, 1), lambda i: (i, 0))
    return pl.pallas_call(
        body,
        grid=(_GRID,),
        in_specs=[specw, specw, specw, spec16, specW, spec1, spec1, spec1,
                  specb],
        out_specs=pl.BlockSpec((G, HID), lambda i: (0, 0)),
        out_shape=jax.ShapeDtypeStruct((G, HID), jnp.float32),
        scratch_shapes=[pltpu.VMEM((G, HID), jnp.float32),
                        pltpu.VMEM((G, HID), jnp.float32)],
    )(sa, sb, yprev, dinv16, wp, b, g, bt, batch2d)


def kernel(x, edge_index, edge_attr, batch,
           W1, b1, g1, bt1, W2, b2, g2, bt2, W3, b3, g3, bt3):
    src = edge_index[0].astype(jnp.int32)
    dst = edge_index[1].astype(jnp.int32)
    xpad = jnp.concatenate(
        [jnp.concatenate([x, jnp.zeros((N, 16 - x.shape[1]), jnp.float32)],
                         axis=1),
         jnp.zeros((NP - N, 16), jnp.float32)], axis=0)
    W1p = jnp.concatenate(
        [W1, jnp.zeros((16 - W1.shape[0], HID), jnp.float32)], axis=0)
    batch2d = batch.astype(jnp.int32).reshape(N, 1)
    r1 = lambda v: v.reshape(1, HID)

    degp = _make_deg_pass()(dst)
    s0, y0pf, dinv16f = _make_p16_pass()(degp, xpad, src, dst)
    y0p = y0pf[:N]
    dinv16 = dinv16f[:N]
    y1 = _tc_layer(s0[:N], s0[NP:NP + N], y0p, dinv16, W1p,
                   r1(b1), r1(g1), r1(bt1))

    s1 = _make_agg_pass(HID, 6, jnp.bfloat16)(y1, src, dst)
    y2 = _tc_layer(s1[:N], s1[NP:NP + N], y1, dinv16, W2,
                   r1(b2), r1(g2), r1(bt2))

    s2 = _make_agg_pass(HID, 6, jnp.bfloat16)(y2, src, dst)
    return _tc_final(s2[:N], s2[NP:NP + N], y2, dinv16, W3,
                     r1(b3), r1(g3), r1(bt3), batch2d)
